# baseline (speedup 1.0000x reference)
import jax
import jax.numpy as jnp
from jax.experimental import pallas as pl
from jax.experimental.pallas import tpu as pltpu


def _tiny(x_ref, ea_ref, o_ref):
    o_ref[...] = jnp.zeros((8, 128), jnp.float32)


def kernel(x, edge_index, edge_attr):
    t = pl.pallas_call(
        _tiny,
        out_shape=jax.ShapeDtypeStruct((8, 128), jnp.float32),
        in_specs=[pl.BlockSpec(memory_space=pl.ANY),
                  pl.BlockSpec(memory_space=pl.ANY)],
    )(x, edge_attr)
    return (x + t[0, 0], edge_attr + t[0, 1])


# speedup vs baseline: 2.7598x; 2.7598x over previous
import jax
import jax.numpy as jnp
from jax.experimental import pallas as pl
from jax.experimental.pallas import tpu as pltpu

_GRID = 50
_XB = 10000 // _GRID
_EB = 320000 // _GRID


def _copy_body(x_ref, ea_ref, xo_ref, eo_ref):
    xo_ref[...] = x_ref[...]
    eo_ref[...] = ea_ref[...]


def kernel(x, edge_index, edge_attr):
    ea_t = edge_attr.T
    x_out, ea_out_t = pl.pallas_call(
        _copy_body,
        grid=(_GRID,),
        out_shape=(
            jax.ShapeDtypeStruct((10000, 128), x.dtype),
            jax.ShapeDtypeStruct((16, 320000), edge_attr.dtype),
        ),
        in_specs=[
            pl.BlockSpec((_XB, 128), lambda i: (i, 0)),
            pl.BlockSpec((16, _EB), lambda i: (0, i)),
        ],
        out_specs=(
            pl.BlockSpec((_XB, 128), lambda i: (i, 0)),
            pl.BlockSpec((16, _EB), lambda i: (0, i)),
        ),
    )(x, ea_t)
    return (x_out, ea_out_t.T)


# speedup vs baseline: 5.4083x; 1.9597x over previous
import jax
import jax.numpy as jnp
from jax.experimental import pallas as pl
from jax.experimental.pallas import tpu as pltpu

_GRID = 10
_XB = 10000 // _GRID
_EB = 320000 // _GRID


def _copy_body(x_ref, ea_ref, xo_ref, eo_ref):
    xo_ref[...] = x_ref[...]
    eo_ref[...] = ea_ref[...]


def kernel(x, edge_index, edge_attr):
    ea_t = edge_attr.T
    x_out, ea_out_t = pl.pallas_call(
        _copy_body,
        grid=(_GRID,),
        out_shape=(
            jax.ShapeDtypeStruct((10000, 128), x.dtype),
            jax.ShapeDtypeStruct((16, 320000), edge_attr.dtype),
        ),
        in_specs=[
            pl.BlockSpec((_XB, 128), lambda i: (i, 0)),
            pl.BlockSpec((16, _EB), lambda i: (0, i)),
        ],
        out_specs=(
            pl.BlockSpec((_XB, 128), lambda i: (i, 0)),
            pl.BlockSpec((16, _EB), lambda i: (0, i)),
        ),
    )(x, ea_t)
    return (x_out, ea_out_t.T)


# speedup vs baseline: 5.8215x; 1.0764x over previous
import jax
import jax.numpy as jnp
from jax.experimental import pallas as pl
from jax.experimental.pallas import tpu as pltpu

_GRID = 5
_XB = 10000 // _GRID
_EB = 320000 // _GRID


def _copy_body(x_ref, ea_ref, xo_ref, eo_ref):
    xo_ref[...] = x_ref[...]
    eo_ref[...] = ea_ref[...]


def kernel(x, edge_index, edge_attr):
    ea_t = edge_attr.T
    x_out, ea_out_t = pl.pallas_call(
        _copy_body,
        grid=(_GRID,),
        out_shape=(
            jax.ShapeDtypeStruct((10000, 128), x.dtype),
            jax.ShapeDtypeStruct((16, 320000), edge_attr.dtype),
        ),
        in_specs=[
            pl.BlockSpec((_XB, 128), lambda i: (i, 0)),
            pl.BlockSpec((16, _EB), lambda i: (0, i)),
        ],
        out_specs=(
            pl.BlockSpec((_XB, 128), lambda i: (i, 0)),
            pl.BlockSpec((16, _EB), lambda i: (0, i)),
        ),
    )(x, ea_t)
    return (x_out, ea_out_t.T)


# speedup vs baseline: 6.4244x; 1.1036x over previous
import jax
import jax.numpy as jnp
from jax.experimental import pallas as pl
from jax.experimental.pallas import tpu as pltpu

_GRID = 2
_XB = 10000 // _GRID
_EB = 320000 // _GRID


def _copy_body(x_ref, ea_ref, xo_ref, eo_ref):
    xo_ref[...] = x_ref[...]
    eo_ref[...] = ea_ref[...]


def kernel(x, edge_index, edge_attr):
    ea_t = edge_attr.T
    x_out, ea_out_t = pl.pallas_call(
        _copy_body,
        grid=(_GRID,),
        out_shape=(
            jax.ShapeDtypeStruct((10000, 128), x.dtype),
            jax.ShapeDtypeStruct((16, 320000), edge_attr.dtype),
        ),
        in_specs=[
            pl.BlockSpec((_XB, 128), lambda i: (i, 0)),
            pl.BlockSpec((16, _EB), lambda i: (0, i)),
        ],
        out_specs=(
            pl.BlockSpec((_XB, 128), lambda i: (i, 0)),
            pl.BlockSpec((16, _EB), lambda i: (0, i)),
        ),
    )(x, ea_t)
    return (x_out, ea_out_t.T)
